# baseline (speedup 1.0000x reference)
import functools

import jax
import jax.numpy as jnp
import numpy as np
from jax import lax
from jax.experimental import pallas as pl
from jax.experimental.pallas import tpu as pltpu
from jax.experimental.pallas import tpu_sc as plsc

B = 64
S = 8192
L = 16
NSB = 16
SB = S // NSB
NCH = SB // L
NC = 2
NS = 16
NW = NC * NS
ROWS_PER_W = B // NW

def _threefry2x32(k1, k2, x0, x1):
    u32 = np.uint32
    k1, k2 = u32(k1), u32(k2)
    x = [x0.astype(u32).copy(), x1.astype(u32).copy()]
    ks = [k1, k2, u32(k1 ^ k2 ^ np.uint32(0x1BD11BDA))]
    rotations = [(13, 15, 26, 6), (17, 29, 16, 24)]

    def rotl(v, d):
        return ((v << u32(d)) | (v >> u32(32 - d))).astype(u32)

    x[0] = (x[0] + ks[0]).astype(u32)
    x[1] = (x[1] + ks[1]).astype(u32)
    for i in range(5):
        for r in rotations[i % 2]:
            x[0] = (x[0] + x[1]).astype(u32)
            x[1] = x[0] ^ rotl(x[1], r)
        x[0] = (x[0] + ks[(i + 1) % 3]).astype(u32)
        x[1] = (x[1] + ks[(i + 2) % 3] + u32(i + 1)).astype(u32)
    return x[0], x[1]


def _u_consts():
    lo = np.arange(B, dtype=np.uint32)
    hi = np.zeros(B, np.uint32)
    k1, k2 = _threefry2x32(0, 42, hi, lo)
    u = np.empty((B, 3), np.float32)
    for i in range(B):
        b1, b2 = _threefry2x32(
            k1[i], k2[i],
            np.zeros(3, np.uint32), np.arange(3, dtype=np.uint32),
        )
        bits = (b1 ^ b2).astype(np.uint32)
        f = ((bits >> np.uint32(9)) | np.uint32(0x3F800000)).view(np.float32)
        u[i] = np.maximum(np.float32(0.0), f - np.float32(1.0))
    return u


def _u_flat():
    up = np.zeros((B, L), np.float32)
    up[:, :3] = _u_consts()
    return up.reshape(B * L)


@functools.cache
def _build_kernel():
    mesh = plsc.VectorSubcoreMesh(
        core_axis_name="c", subcore_axis_name="s", num_cores=NC, num_subcores=NS
    )
    return pl.kernel(
        _sample_body,
        out_type=(
            jax.ShapeDtypeStruct((B, S), jnp.int32),
            jax.ShapeDtypeStruct((B, S), jnp.int32),
        ),
        mesh=mesh,
        scratch_types=[
            pltpu.VMEM((S,), jnp.int32),
            pltpu.VMEM((S,), jnp.int32),
            pltpu.VMEM((S,), jnp.int32),
            pltpu.VMEM((S,), jnp.int32),
            pltpu.VMEM((2 * L,), jnp.float32),
            pltpu.SemaphoreType.DMA,
            pltpu.SemaphoreType.DMA,
            pltpu.SemaphoreType.DMA,
        ],
        compiler_params=pltpu.CompilerParams(
            needs_layout_passes=False, use_tc_tiling_on_sc=True
        ),
    )


def _find_indices(mask_v, u_v, ubase, iota):
    zeros16 = jnp.zeros((L,), jnp.int32)

    @plsc.parallel_loop(0, SB, step=8, carry=zeros16)
    def svec(i, acc):
        g = zeros16
        for k in range(8):
            off = (iota + (i + k)) & (SB - 1)
            g = g + plsc.load_gather(mask_v, [iota * SB + off])
        return acc + g

    s = jnp.sum(svec)
    sfv = jnp.full((L,), s, jnp.int32).astype(jnp.float32)
    onev = jnp.full((L,), jnp.float32(1.0))
    c1v = onev / jnp.maximum(sfv, onev)
    stv = sfv * c1v
    pcum = plsc.cumsum(svec)

    idxs = []
    for t in range(3):
        uvec = u_v[pl.ds(ubase, L)]
        ut = jnp.sum(jnp.where(iota == t, uvec, jnp.float32(0.0)))
        rtv = stv * (onev - ut)
        est = jnp.max((rtv * sfv).astype(jnp.int32))
        base = jnp.maximum(est - 8, 0)
        ks = base + 1 + iota
        g = ks.astype(jnp.float32) * c1v
        cnt = jnp.sum(jnp.where((g < rtv) & (ks <= s), 1, 0))
        m = base + cnt + 1

        bt = jnp.min(jnp.where(pcum >= m, iota, NSB))
        bt = jnp.minimum(bt, NSB - 1)
        pbefore = jnp.sum(jnp.where(iota < bt, svec, 0))
        m2 = m - pbefore

        @plsc.parallel_loop(0, 32, step=8, carry=zeros16)
        def ssum(off, acc):
            g2 = zeros16
            for k in range(8):
                o2 = (iota + (off + k)) & 31
                g2 = g2 + plsc.load_gather(mask_v, [bt * SB + iota * 32 + o2])
            return acc + g2

        scum = plsc.cumsum(ssum)
        ct = jnp.min(jnp.where(scum >= m2, iota, L))
        ct = jnp.minimum(ct, L - 1)
        pb2 = jnp.sum(jnp.where(iota < ct, ssum, 0))
        m3 = m2 - pb2

        cbase = bt * SB + ct * 32
        v0 = plsc.load_gather(mask_v, [cbase + iota])
        v1 = plsc.load_gather(mask_v, [cbase + L + iota])
        lc0 = plsc.cumsum(v0)
        lc1 = plsc.cumsum(v1)
        tot0 = jnp.max(lc0)
        lane0 = jnp.min(jnp.where(lc0 >= m3, iota, L))
        lane1 = jnp.min(jnp.where(lc1 >= (m3 - tot0), iota, L))
        idx = cbase + jnp.where(
            m3 <= tot0,
            jnp.minimum(lane0, L - 1),
            L + jnp.minimum(lane1, L - 1),
        )
        idxs.append(jnp.where(s > 0, idx, 0))

    iv = (
        jnp.where(iota == 0, idxs[0], 0)
        + jnp.where(iota == 1, idxs[1], 0)
        + jnp.where(iota == 2, idxs[2], 0)
    )
    return iv


def _sample_body(
    mask_hbm, ids_hbm, u_hbm, ids_out, out_hbm,
    m_a, m_b, r_a, r_b, u_v, sem_in, sem_out, sem_ids,
):
    wid = lax.axis_index("s") * NC + lax.axis_index("c")
    iota = lax.iota(jnp.int32, L)
    zeros16 = jnp.zeros((L,), jnp.int32)
    row0 = wid * ROWS_PER_W

    cp_a = pltpu.make_async_copy(mask_hbm.at[row0], m_a, sem_in)
    cp_a.start()
    cp_b = pltpu.make_async_copy(mask_hbm.at[row0 + 1], m_b, sem_in)
    cp_b.start()
    cpi_a = pltpu.make_async_copy(ids_hbm.at[row0], ids_out.at[row0], sem_ids)
    cpi_a.start()
    cpi_b = pltpu.make_async_copy(
        ids_hbm.at[row0 + 1], ids_out.at[row0 + 1], sem_ids
    )
    cpi_b.start()
    pltpu.sync_copy(u_hbm.at[pl.ds(row0 * L, 2 * L)], u_v)

    @plsc.parallel_loop(0, S // L, unroll=8)
    def _zinit(i):
        r_a[pl.ds(i * L, L)] = zeros16
        r_b[pl.ds(i * L, L)] = zeros16

    msk3 = iota < 3
    ones16 = jnp.ones((L,), jnp.int32)

    cp_a.wait()
    iv_a = _find_indices(m_a, u_v, 0, iota)
    plsc.store_scatter(r_a, [iv_a], ones16, mask=msk3)
    out_a = pltpu.make_async_copy(r_a, out_hbm.at[row0], sem_out)
    out_a.start()

    cp_b.wait()
    iv_b = _find_indices(m_b, u_v, L, iota)
    plsc.store_scatter(r_b, [iv_b], ones16, mask=msk3)
    out_b = pltpu.make_async_copy(r_b, out_hbm.at[row0 + 1], sem_out)
    out_b.start()

    out_a.wait()
    out_b.wait()
    cpi_a.wait()
    cpi_b.wait()


def kernel(input_ids, my_attention_mask):
    assert my_attention_mask.shape == (B, S)
    u = jnp.asarray(_u_flat())
    ids_out, new_mask = _build_kernel()(my_attention_mask, input_ids, u)
    return (ids_out, new_mask)


# speedup vs baseline: 3.2260x; 3.2260x over previous
import functools

import jax
import jax.numpy as jnp
import numpy as np
from jax import lax
from jax.experimental import pallas as pl
from jax.experimental.pallas import tpu as pltpu
from jax.experimental.pallas import tpu_sc as plsc

B = 64
S = 8192
L = 16
NSB = 16
SB = S // NSB
NCH = SB // L
NC = 2
NS = 16
NW = NC * NS
ROWS_PER_W = B // NW

def _threefry2x32(k1, k2, x0, x1):
    u32 = np.uint32
    k1, k2 = u32(k1), u32(k2)
    x = [x0.astype(u32).copy(), x1.astype(u32).copy()]
    ks = [k1, k2, u32(k1 ^ k2 ^ np.uint32(0x1BD11BDA))]
    rotations = [(13, 15, 26, 6), (17, 29, 16, 24)]

    def rotl(v, d):
        return ((v << u32(d)) | (v >> u32(32 - d))).astype(u32)

    x[0] = (x[0] + ks[0]).astype(u32)
    x[1] = (x[1] + ks[1]).astype(u32)
    for i in range(5):
        for r in rotations[i % 2]:
            x[0] = (x[0] + x[1]).astype(u32)
            x[1] = x[0] ^ rotl(x[1], r)
        x[0] = (x[0] + ks[(i + 1) % 3]).astype(u32)
        x[1] = (x[1] + ks[(i + 2) % 3] + u32(i + 1)).astype(u32)
    return x[0], x[1]


def _u_consts():
    lo = np.arange(B, dtype=np.uint32)
    hi = np.zeros(B, np.uint32)
    k1, k2 = _threefry2x32(0, 42, hi, lo)
    u = np.empty((B, 3), np.float32)
    for i in range(B):
        b1, b2 = _threefry2x32(
            k1[i], k2[i],
            np.zeros(3, np.uint32), np.arange(3, dtype=np.uint32),
        )
        bits = (b1 ^ b2).astype(np.uint32)
        f = ((bits >> np.uint32(9)) | np.uint32(0x3F800000)).view(np.float32)
        u[i] = np.maximum(np.float32(0.0), f - np.float32(1.0))
    return u


def _u_flat():
    up = np.zeros((B, L), np.float32)
    up[:, :3] = _u_consts()
    return up.reshape(B * L)


@functools.cache
def _build_kernel():
    mesh = plsc.VectorSubcoreMesh(
        core_axis_name="c", subcore_axis_name="s", num_cores=NC, num_subcores=NS
    )
    return pl.kernel(
        _sample_body,
        out_type=jax.ShapeDtypeStruct((B, S), jnp.int32),
        mesh=mesh,
        scratch_types=[
            pltpu.VMEM((S,), jnp.int32),
            pltpu.VMEM((S,), jnp.int32),
            pltpu.VMEM((S,), jnp.int32),
            pltpu.VMEM((S,), jnp.int32),
            pltpu.VMEM((2 * L,), jnp.float32),
            pltpu.SemaphoreType.DMA,
            pltpu.SemaphoreType.DMA,
        ],
        compiler_params=pltpu.CompilerParams(
            needs_layout_passes=False, use_tc_tiling_on_sc=True
        ),
    )


def _find_indices(mask_v, u_v, ubase, iota):
    zeros16 = jnp.zeros((L,), jnp.int32)

    @plsc.parallel_loop(0, SB, step=8, carry=zeros16)
    def svec(i, acc):
        g = zeros16
        for k in range(8):
            off = (iota + (i + k)) & (SB - 1)
            g = g + plsc.load_gather(mask_v, [iota * SB + off])
        return acc + g

    s = jnp.sum(svec)
    sfv = jnp.full((L,), s, jnp.int32).astype(jnp.float32)
    onev = jnp.full((L,), jnp.float32(1.0))
    c1v = onev / jnp.maximum(sfv, onev)
    stv = sfv * c1v
    pcum = plsc.cumsum(svec)

    idxs = []
    for t in range(3):
        uvec = u_v[pl.ds(ubase, L)]
        ut = jnp.sum(jnp.where(iota == t, uvec, jnp.float32(0.0)))
        rtv = stv * (onev - ut)
        est = jnp.max((rtv * sfv).astype(jnp.int32))
        base = jnp.maximum(est - 8, 0)
        ks = base + 1 + iota
        g = ks.astype(jnp.float32) * c1v
        cnt = jnp.sum(jnp.where((g < rtv) & (ks <= s), 1, 0))
        m = base + cnt + 1

        bt = jnp.min(jnp.where(pcum >= m, iota, NSB))
        bt = jnp.minimum(bt, NSB - 1)
        pbefore = jnp.sum(jnp.where(iota < bt, svec, 0))
        m2 = m - pbefore

        @plsc.parallel_loop(0, 32, step=8, carry=zeros16)
        def ssum(off, acc):
            g2 = zeros16
            for k in range(8):
                o2 = (iota + (off + k)) & 31
                g2 = g2 + plsc.load_gather(mask_v, [bt * SB + iota * 32 + o2])
            return acc + g2

        scum = plsc.cumsum(ssum)
        ct = jnp.min(jnp.where(scum >= m2, iota, L))
        ct = jnp.minimum(ct, L - 1)
        pb2 = jnp.sum(jnp.where(iota < ct, ssum, 0))
        m3 = m2 - pb2

        cbase = bt * SB + ct * 32
        v0 = plsc.load_gather(mask_v, [cbase + iota])
        v1 = plsc.load_gather(mask_v, [cbase + L + iota])
        lc0 = plsc.cumsum(v0)
        lc1 = plsc.cumsum(v1)
        tot0 = jnp.max(lc0)
        lane0 = jnp.min(jnp.where(lc0 >= m3, iota, L))
        lane1 = jnp.min(jnp.where(lc1 >= (m3 - tot0), iota, L))
        idx = cbase + jnp.where(
            m3 <= tot0,
            jnp.minimum(lane0, L - 1),
            L + jnp.minimum(lane1, L - 1),
        )
        idxs.append(jnp.where(s > 0, idx, 0))

    iv = (
        jnp.where(iota == 0, idxs[0], 0)
        + jnp.where(iota == 1, idxs[1], 0)
        + jnp.where(iota == 2, idxs[2], 0)
    )
    return iv


def _sample_body(mask_hbm, u_hbm, out_hbm, m_a, m_b, r_a, r_b, u_v, sem_in, sem_out):
    wid = lax.axis_index("s") * NC + lax.axis_index("c")
    iota = lax.iota(jnp.int32, L)
    zeros16 = jnp.zeros((L,), jnp.int32)
    row0 = wid * ROWS_PER_W

    cp_a = pltpu.make_async_copy(mask_hbm.at[row0], m_a, sem_in)
    cp_a.start()
    cp_b = pltpu.make_async_copy(mask_hbm.at[row0 + 1], m_b, sem_in)
    cp_b.start()
    pltpu.sync_copy(u_hbm.at[pl.ds(row0 * L, 2 * L)], u_v)

    @plsc.parallel_loop(0, S // L, unroll=8)
    def _zinit(i):
        r_a[pl.ds(i * L, L)] = zeros16
        r_b[pl.ds(i * L, L)] = zeros16

    msk3 = iota < 3
    ones16 = jnp.ones((L,), jnp.int32)

    cp_a.wait()
    iv_a = _find_indices(m_a, u_v, 0, iota)
    plsc.store_scatter(r_a, [iv_a], ones16, mask=msk3)
    out_a = pltpu.make_async_copy(r_a, out_hbm.at[row0], sem_out)
    out_a.start()

    cp_b.wait()
    iv_b = _find_indices(m_b, u_v, L, iota)
    plsc.store_scatter(r_b, [iv_b], ones16, mask=msk3)
    out_b = pltpu.make_async_copy(r_b, out_hbm.at[row0 + 1], sem_out)
    out_b.start()

    out_a.wait()
    out_b.wait()


def kernel(input_ids, my_attention_mask):
    assert my_attention_mask.shape == (B, S)
    u = jnp.asarray(_u_flat())
    new_mask = _build_kernel()(my_attention_mask, u)
    return (input_ids, new_mask)


# speedup vs baseline: 3.3006x; 1.0231x over previous
import functools

import jax
import jax.numpy as jnp
import numpy as np
from jax import lax
from jax.experimental import pallas as pl
from jax.experimental.pallas import tpu as pltpu
from jax.experimental.pallas import tpu_sc as plsc

B = 64
S = 8192
L = 16
NSB = 16
SB = S // NSB
NCH = SB // L
NC = 2
NS = 16
NW = NC * NS
ROWS_PER_W = B // NW

def _threefry2x32(k1, k2, x0, x1):
    u32 = np.uint32
    k1, k2 = u32(k1), u32(k2)
    x = [x0.astype(u32).copy(), x1.astype(u32).copy()]
    ks = [k1, k2, u32(k1 ^ k2 ^ np.uint32(0x1BD11BDA))]
    rotations = [(13, 15, 26, 6), (17, 29, 16, 24)]

    def rotl(v, d):
        return ((v << u32(d)) | (v >> u32(32 - d))).astype(u32)

    x[0] = (x[0] + ks[0]).astype(u32)
    x[1] = (x[1] + ks[1]).astype(u32)
    for i in range(5):
        for r in rotations[i % 2]:
            x[0] = (x[0] + x[1]).astype(u32)
            x[1] = x[0] ^ rotl(x[1], r)
        x[0] = (x[0] + ks[(i + 1) % 3]).astype(u32)
        x[1] = (x[1] + ks[(i + 2) % 3] + u32(i + 1)).astype(u32)
    return x[0], x[1]


def _u_consts():
    lo = np.arange(B, dtype=np.uint32)
    hi = np.zeros(B, np.uint32)
    k1, k2 = _threefry2x32(0, 42, hi, lo)
    u = np.empty((B, 3), np.float32)
    for i in range(B):
        b1, b2 = _threefry2x32(
            k1[i], k2[i],
            np.zeros(3, np.uint32), np.arange(3, dtype=np.uint32),
        )
        bits = (b1 ^ b2).astype(np.uint32)
        f = ((bits >> np.uint32(9)) | np.uint32(0x3F800000)).view(np.float32)
        u[i] = np.maximum(np.float32(0.0), f - np.float32(1.0))
    return u


def _u_flat():
    up = np.zeros((B, L), np.float32)
    up[:, :3] = _u_consts()
    return up.reshape(B * L)


@functools.cache
def _build_kernel():
    mesh = plsc.VectorSubcoreMesh(
        core_axis_name="c", subcore_axis_name="s", num_cores=NC, num_subcores=NS
    )
    return pl.kernel(
        _sample_body,
        out_type=jax.ShapeDtypeStruct((B, S), jnp.int32),
        mesh=mesh,
        scratch_types=[
            pltpu.VMEM((S,), jnp.int32),
            pltpu.VMEM((S,), jnp.int32),
            pltpu.VMEM((S,), jnp.int32),
            pltpu.VMEM((S,), jnp.int32),
            pltpu.VMEM((2 * L,), jnp.float32),
            pltpu.SemaphoreType.DMA,
            pltpu.SemaphoreType.DMA,
        ],
        compiler_params=pltpu.CompilerParams(
            needs_layout_passes=False, use_tc_tiling_on_sc=True
        ),
    )


def _find_indices(mask_v, u_v, ubase, iota):
    zeros16 = jnp.zeros((L,), jnp.int32)

    @plsc.parallel_loop(0, SB, step=8, carry=zeros16)
    def svec(i, acc):
        g = zeros16
        for k in range(8):
            off = (iota + (i + k)) & (SB - 1)
            g = g + plsc.load_gather(mask_v, [iota * SB + off])
        return acc + g

    s = jnp.sum(svec)
    sfv = jnp.full((L,), s, jnp.int32).astype(jnp.float32)
    onev = jnp.full((L,), jnp.float32(1.0))
    c1v = onev / jnp.maximum(sfv, onev)
    stv = sfv * c1v
    pcum = plsc.cumsum(svec)

    uvec = u_v[pl.ds(ubase, L)]

    def one_target(t, iv_acc):
        ut = jnp.sum(jnp.where(iota == t, uvec, jnp.float32(0.0)))
        rtv = stv * (onev - ut)
        est = jnp.max((rtv * sfv).astype(jnp.int32))
        base = jnp.maximum(est - 8, 0)
        ks = base + 1 + iota
        g = ks.astype(jnp.float32) * c1v
        cnt = jnp.sum(jnp.where((g < rtv) & (ks <= s), 1, 0))
        m = base + cnt + 1

        bt = jnp.min(jnp.where(pcum >= m, iota, NSB))
        bt = jnp.minimum(bt, NSB - 1)
        pbefore = jnp.sum(jnp.where(iota < bt, svec, 0))
        m2 = m - pbefore

        @plsc.parallel_loop(0, 32, step=8, carry=zeros16)
        def ssum(off, acc):
            g2 = zeros16
            for k in range(8):
                o2 = (iota + (off + k)) & 31
                g2 = g2 + plsc.load_gather(mask_v, [bt * SB + iota * 32 + o2])
            return acc + g2

        scum = plsc.cumsum(ssum)
        ct = jnp.min(jnp.where(scum >= m2, iota, L))
        ct = jnp.minimum(ct, L - 1)
        pb2 = jnp.sum(jnp.where(iota < ct, ssum, 0))
        m3 = m2 - pb2

        cbase = bt * SB + ct * 32
        v0 = plsc.load_gather(mask_v, [cbase + iota])
        v1 = plsc.load_gather(mask_v, [cbase + L + iota])
        lc0 = plsc.cumsum(v0)
        lc1 = plsc.cumsum(v1)
        tot0 = jnp.max(lc0)
        lane0 = jnp.min(jnp.where(lc0 >= m3, iota, L))
        lane1 = jnp.min(jnp.where(lc1 >= (m3 - tot0), iota, L))
        idx = cbase + jnp.where(
            m3 <= tot0,
            jnp.minimum(lane0, L - 1),
            L + jnp.minimum(lane1, L - 1),
        )
        idx = jnp.where(s > 0, idx, 0)
        return jnp.where(iota == t, idx, iv_acc)

    return lax.fori_loop(0, 3, one_target, jnp.zeros((L,), jnp.int32))


def _sample_body(mask_hbm, u_hbm, out_hbm, m_a, m_b, r_a, r_b, u_v, sem_in, sem_out):
    wid = lax.axis_index("s") * NC + lax.axis_index("c")
    iota = lax.iota(jnp.int32, L)
    zeros16 = jnp.zeros((L,), jnp.int32)
    row0 = wid * ROWS_PER_W

    cp_a = pltpu.make_async_copy(mask_hbm.at[row0], m_a, sem_in)
    cp_a.start()
    cp_b = pltpu.make_async_copy(mask_hbm.at[row0 + 1], m_b, sem_in)
    cp_b.start()
    pltpu.sync_copy(u_hbm.at[pl.ds(row0 * L, 2 * L)], u_v)

    @plsc.parallel_loop(0, S // L, unroll=8)
    def _zinit(i):
        r_a[pl.ds(i * L, L)] = zeros16
        r_b[pl.ds(i * L, L)] = zeros16

    msk3 = iota < 3
    ones16 = jnp.ones((L,), jnp.int32)

    cp_a.wait()
    iv_a = _find_indices(m_a, u_v, 0, iota)
    plsc.store_scatter(r_a, [iv_a], ones16, mask=msk3)
    out_a = pltpu.make_async_copy(r_a, out_hbm.at[row0], sem_out)
    out_a.start()

    cp_b.wait()
    iv_b = _find_indices(m_b, u_v, L, iota)
    plsc.store_scatter(r_b, [iv_b], ones16, mask=msk3)
    out_b = pltpu.make_async_copy(r_b, out_hbm.at[row0 + 1], sem_out)
    out_b.start()

    out_a.wait()
    out_b.wait()


def kernel(input_ids, my_attention_mask):
    assert my_attention_mask.shape == (B, S)
    u = jnp.asarray(_u_flat())
    new_mask = _build_kernel()(my_attention_mask, u)
    return (input_ids, new_mask)
